# baseline (speedup 1.0000x reference)
import jax
import jax.numpy as jnp
from jax.experimental import pallas as pl
from jax.experimental.pallas import tpu as pltpu

N = 10000
IN_CH = 512
NUM_RES = 4
HID = 1024
BN = 1000
W = IN_CH * NUM_RES


def _head_kernel(x_ref, w_int_ref, W1e_ref, b1_ref, w2_ref, bias_ref, out_ref):
    x = x_ref[...]
    lin = jnp.sum(x * w_int_ref[...], axis=1, keepdims=True)
    h = jnp.dot(x.astype(jnp.bfloat16), W1e_ref[...],
                preferred_element_type=jnp.float32)
    h = h + b1_ref[...]
    h = h * jax.nn.sigmoid(h)
    mlp = jnp.sum(h * w2_ref[...], axis=1, keepdims=True)
    out_ref[...] = lin + mlp + bias_ref[...]


def kernel(embedding_0, W_lin, b_lin, W1, b1, W2, b2):
    x2d = embedding_0.reshape(N, W)
    w_full = jnp.concatenate(
        [W_lin[:, :, 0], jnp.zeros((1, IN_CH), jnp.float32)], axis=0)
    w_int = w_full.T.reshape(1, W)
    W1e = jnp.zeros((IN_CH, NUM_RES, HID), jnp.float32)
    W1e = W1e.at[:, NUM_RES - 1, :].set(W1).reshape(W, HID).astype(jnp.bfloat16)
    bias = (jnp.sum(b_lin) + b2[0]).reshape(1, 1)
    out = pl.pallas_call(
        _head_kernel,
        grid=(N // BN,),
        in_specs=[
            pl.BlockSpec((BN, W), lambda i: (i, 0)),
            pl.BlockSpec((1, W), lambda i: (0, 0)),
            pl.BlockSpec((W, HID), lambda i: (0, 0)),
            pl.BlockSpec((1, HID), lambda i: (0, 0)),
            pl.BlockSpec((1, HID), lambda i: (0, 0)),
            pl.BlockSpec((1, 1), lambda i: (0, 0)),
        ],
        out_specs=pl.BlockSpec((BN, 1), lambda i: (i, 0)),
        out_shape=jax.ShapeDtypeStruct((N, 1), jnp.float32),
        compiler_params=pltpu.CompilerParams(dimension_semantics=("parallel",)),
    )(x2d, w_int, W1e, b1.reshape(1, HID), W2.reshape(1, HID), bias)
    return out.reshape(N)


# speedup vs baseline: 2.1680x; 2.1680x over previous
import jax
import jax.numpy as jnp
from jax.experimental import pallas as pl
from jax.experimental.pallas import tpu as pltpu

N = 10000
IN_CH = 512
NUM_RES = 4
HID = 1024
BN = 1000
NSTRIPE = IN_CH // 128


def _head_kernel(x0_ref, x1_ref, x2_ref, x3_ref, wl_ref, W1_ref, b1_ref,
                 w2_ref, bias_ref, out_ref):
    parts = (x0_ref, x1_ref, x2_ref, x3_ref)
    xlast = jnp.concatenate(
        [p[pl.ds(NUM_RES - 1, BN, NUM_RES), :] for p in parts], axis=1)
    h = jnp.dot(xlast.astype(jnp.bfloat16), W1_ref[...],
                preferred_element_type=jnp.float32)
    h = h + b1_ref[...]
    h = h * jax.nn.sigmoid(h)
    acc = jnp.sum(h * w2_ref[...], axis=1, keepdims=True)
    for r in range(NUM_RES - 1):
        for k, p in enumerate(parts):
            xr = p[pl.ds(r, BN, NUM_RES), :]
            wseg = wl_ref[:, r * IN_CH + k * 128:r * IN_CH + (k + 1) * 128]
            acc = acc + jnp.sum(xr * wseg, axis=1, keepdims=True)
    out_ref[...] = acc + bias_ref[...]


def kernel(embedding_0, W_lin, b_lin, W1, b1, W2, b2):
    xt = jnp.transpose(jnp.squeeze(embedding_0, 2), (0, 2, 1))
    xt = xt.reshape(N * NUM_RES, IN_CH)
    wl = W_lin[:, :, 0].reshape(1, (NUM_RES - 1) * IN_CH)
    bias = (jnp.sum(b_lin) + b2[0]).reshape(1, 1)

    def stripe_spec(k):
        return pl.BlockSpec((NUM_RES * BN, 128), lambda i, k=k: (i, k))

    out = pl.pallas_call(
        _head_kernel,
        grid=(N // BN,),
        in_specs=[stripe_spec(k) for k in range(NSTRIPE)] + [
            pl.BlockSpec((1, (NUM_RES - 1) * IN_CH), lambda i: (0, 0)),
            pl.BlockSpec((IN_CH, HID), lambda i: (0, 0)),
            pl.BlockSpec((1, HID), lambda i: (0, 0)),
            pl.BlockSpec((1, HID), lambda i: (0, 0)),
            pl.BlockSpec((1, 1), lambda i: (0, 0)),
        ],
        out_specs=pl.BlockSpec((BN, 1), lambda i: (i, 0)),
        out_shape=jax.ShapeDtypeStruct((N, 1), jnp.float32),
        compiler_params=pltpu.CompilerParams(dimension_semantics=("parallel",)),
    )(xt, xt, xt, xt, wl, W1.astype(jnp.bfloat16), b1.reshape(1, HID),
      W2.reshape(1, HID), bias)
    return out.reshape(N)


# speedup vs baseline: 4.7800x; 2.2048x over previous
import jax
import jax.numpy as jnp
from jax.experimental import pallas as pl
from jax.experimental.pallas import tpu as pltpu

N = 10000
IN_CH = 512
NUM_RES = 4
HID = 1024
BN = 1000
NSTRIPE = IN_CH // 128
RPN = NUM_RES * NSTRIPE


def _head_kernel(x_ref, wl_ref, W1_ref, b1_ref, w2_ref, bias_ref, out_ref):
    xlast = jnp.concatenate(
        [x_ref[pl.ds(NUM_RES * t + NUM_RES - 1, BN, RPN), :]
         for t in range(NSTRIPE)], axis=1)
    h = jnp.dot(xlast.astype(jnp.bfloat16), W1_ref[...],
                preferred_element_type=jnp.float32)
    h = h + b1_ref[...]
    h = h * jax.nn.sigmoid(h)
    acc = jnp.sum(h * w2_ref[...], axis=1, keepdims=True)
    for r in range(NUM_RES - 1):
        for t in range(NSTRIPE):
            xr = x_ref[pl.ds(NUM_RES * t + r, BN, RPN), :]
            wseg = wl_ref[:, r * IN_CH + t * 128:r * IN_CH + (t + 1) * 128]
            acc = acc + jnp.sum(xr * wseg, axis=1, keepdims=True)
    out_ref[...] = acc + bias_ref[...]


def kernel(embedding_0, W_lin, b_lin, W1, b1, W2, b2):
    x = jnp.squeeze(embedding_0, 2)
    x = x.reshape(N, NSTRIPE, 128, NUM_RES)
    x = jnp.transpose(x, (0, 1, 3, 2))
    x = x.reshape(N * RPN, 128)
    wl = W_lin[:, :, 0].reshape(1, (NUM_RES - 1) * IN_CH)
    bias = (jnp.sum(b_lin) + b2[0]).reshape(1, 1)

    out = pl.pallas_call(
        _head_kernel,
        grid=(N // BN,),
        in_specs=[
            pl.BlockSpec((RPN * BN, 128), lambda i: (i, 0)),
            pl.BlockSpec((1, (NUM_RES - 1) * IN_CH), lambda i: (0, 0)),
            pl.BlockSpec((IN_CH, HID), lambda i: (0, 0)),
            pl.BlockSpec((1, HID), lambda i: (0, 0)),
            pl.BlockSpec((1, HID), lambda i: (0, 0)),
            pl.BlockSpec((1, 1), lambda i: (0, 0)),
        ],
        out_specs=pl.BlockSpec((BN, 1), lambda i: (i, 0)),
        out_shape=jax.ShapeDtypeStruct((N, 1), jnp.float32),
        compiler_params=pltpu.CompilerParams(dimension_semantics=("parallel",)),
    )(x, wl, W1.astype(jnp.bfloat16), b1.reshape(1, HID),
      W2.reshape(1, HID), bias)
    return out.reshape(N)
